# baseline (speedup 1.0000x reference)
import functools

import jax
import jax.numpy as jnp
from jax import lax
from jax.experimental import pallas as pl
from jax.experimental.pallas import tpu as pltpu
from jax.experimental.pallas import tpu_sc as plsc

_INFO = plsc.get_sparse_core_info()
_NC, _NS = _INFO.num_cores, _INFO.num_subcores
_NW = _NC * _NS


def _embed_gather(table_hbm, idx_hbm, out_hbm,
                  idx_v0, idx_v1, rows_v0, rows_v1,
                  sem_g0, sem_g1, sem_s0, sem_s1,
                  *, b_per_w, chunk):
    wid = lax.axis_index("s") * _NC + lax.axis_index("c")
    base_w = wid * b_per_w
    n_chunks = b_per_w // chunk
    idx_v = (idx_v0, idx_v1)
    rows_v = (rows_v0, rows_v1)
    sem_g = (sem_g0, sem_g1)
    sem_s = (sem_s0, sem_s1)

    def chunk_slice(g):
        return pl.ds(pl.multiple_of(base_w + g * chunk, 8), chunk)

    def body(i, carry):
        for b in range(2):
            g = 2 * i + b

            @pl.when(i >= 1)
            def _wait_prev_store():
                pltpu.make_async_copy(
                    rows_v[b], out_hbm.at[chunk_slice(g)], sem_s[b]
                ).wait()

            pltpu.sync_copy(idx_hbm.at[chunk_slice(g)], idx_v[b])
            pltpu.async_copy(table_hbm.at[idx_v[b]], rows_v[b], sem_g[b])
        for b in range(2):
            g = 2 * i + b
            pltpu.make_async_copy(
                table_hbm.at[idx_v[b]], rows_v[b], sem_g[b]
            ).wait()
            pltpu.async_copy(rows_v[b], out_hbm.at[chunk_slice(g)], sem_s[b])
        return carry

    lax.fori_loop(0, n_chunks // 2, body, 0)
    for b in range(2):
        g = n_chunks - 2 + b
        pltpu.make_async_copy(
            rows_v[b], out_hbm.at[chunk_slice(g)], sem_s[b]
        ).wait()


def kernel(input_ids, embed_table):
    B, S = input_ids.shape
    V, D = embed_table.shape
    n = B * S
    assert n % _NW == 0
    b_per_w = n // _NW
    chunk = 800
    assert b_per_w % (2 * chunk) == 0

    idx_flat = input_ids.reshape(n)
    one = lax.optimization_barrier(jnp.float32(1.0))
    table_sc = embed_table * one

    mesh = plsc.VectorSubcoreMesh(core_axis_name="c", subcore_axis_name="s")
    k = pl.kernel(
        functools.partial(_embed_gather, b_per_w=b_per_w, chunk=chunk),
        mesh=mesh,
        out_type=jax.ShapeDtypeStruct((n, D), jnp.float32),
        scratch_types=[
            pltpu.VMEM((chunk,), jnp.int32),
            pltpu.VMEM((chunk,), jnp.int32),
            pltpu.VMEM((chunk, D), jnp.float32),
            pltpu.VMEM((chunk, D), jnp.float32),
            pltpu.SemaphoreType.DMA,
            pltpu.SemaphoreType.DMA,
            pltpu.SemaphoreType.DMA,
            pltpu.SemaphoreType.DMA,
        ],
        compiler_params=pltpu.CompilerParams(use_tc_tiling_on_sc=False),
    )
    out = k(table_sc, idx_flat)
    return out.reshape(B, S, D) * one


# speedup vs baseline: 1.4695x; 1.4695x over previous
import functools

import jax
import jax.numpy as jnp
from jax import lax
from jax.experimental import pallas as pl
from jax.experimental.pallas import tpu as pltpu
from jax.experimental.pallas import tpu_sc as plsc

_INFO = plsc.get_sparse_core_info()
_NC, _NS = _INFO.num_cores, _INFO.num_subcores
_NW = _NC * _NS


def _embed_gather(table_hbm, idx_hbm, out_hbm,
                  idx_v0, idx_v1, r128_0, r128_1, r64_0, r64_1,
                  sem_g0, sem_g1, sem_s0, sem_s1,
                  *, b_per_w, chunk):
    wid = lax.axis_index("s") * _NC + lax.axis_index("c")
    base_w = wid * b_per_w
    n_chunks = b_per_w // chunk
    idx_v = (idx_v0, idx_v1)
    r128 = (r128_0, r128_1)
    r64 = (r64_0, r64_1)
    sem_g = (sem_g0, sem_g1)
    sem_s = (sem_s0, sem_s1)

    def chunk_slice(g):
        return pl.ds(pl.multiple_of(base_w + g * chunk, 8), chunk)

    def compact(b):
        def row(i, carry):
            for j in range(4):
                sl = pl.ds(j * 16, 16)
                r64[b][i, sl] = r128[b][i, sl]
            return carry
        lax.fori_loop(0, chunk, row, 0, unroll=8)

    def body(i, carry):
        for b in range(2):
            g = 2 * i + b

            @pl.when(i >= 1)
            def _wait_prev_store():
                pltpu.make_async_copy(
                    r64[b], out_hbm.at[chunk_slice(g), :], sem_s[b]
                ).wait()

            pltpu.sync_copy(idx_hbm.at[chunk_slice(g)], idx_v[b])
            pltpu.async_copy(table_hbm.at[idx_v[b]], r128[b], sem_g[b])
        for b in range(2):
            g = 2 * i + b
            pltpu.make_async_copy(
                table_hbm.at[idx_v[b]], r128[b], sem_g[b]
            ).wait()
            compact(b)
            pltpu.async_copy(r64[b], out_hbm.at[chunk_slice(g), :], sem_s[b])
        return carry

    lax.fori_loop(0, n_chunks // 2, body, 0)
    for b in range(2):
        g = n_chunks - 2 + b
        pltpu.make_async_copy(
            r64[b], out_hbm.at[chunk_slice(g), :], sem_s[b]
        ).wait()


def kernel(input_ids, embed_table):
    B, S = input_ids.shape
    V, D = embed_table.shape
    n = B * S
    assert n % _NW == 0
    b_per_w = n // _NW
    chunk = 200
    assert b_per_w % (2 * chunk) == 0

    idx_flat = input_ids.reshape(n)
    one = lax.optimization_barrier(jnp.float32(1.0))
    padded = jnp.pad(embed_table, ((0, 0), (0, 128 - D))) * one

    mesh = plsc.VectorSubcoreMesh(core_axis_name="c", subcore_axis_name="s")
    k = pl.kernel(
        functools.partial(_embed_gather, b_per_w=b_per_w, chunk=chunk),
        mesh=mesh,
        out_type=jax.ShapeDtypeStruct((n, D), jnp.float32),
        scratch_types=[
            pltpu.VMEM((chunk,), jnp.int32),
            pltpu.VMEM((chunk,), jnp.int32),
            pltpu.VMEM((chunk, 128), jnp.float32),
            pltpu.VMEM((chunk, 128), jnp.float32),
            pltpu.VMEM((chunk, D), jnp.float32),
            pltpu.VMEM((chunk, D), jnp.float32),
            pltpu.SemaphoreType.DMA,
            pltpu.SemaphoreType.DMA,
            pltpu.SemaphoreType.DMA,
            pltpu.SemaphoreType.DMA,
        ],
        compiler_params=pltpu.CompilerParams(use_tc_tiling_on_sc=True),
    )
    out = k(padded, idx_flat)
    return out.reshape(B, S, D)


# speedup vs baseline: 1.5045x; 1.0238x over previous
import functools

import jax
import jax.numpy as jnp
from jax import lax
from jax.experimental import pallas as pl
from jax.experimental.pallas import tpu as pltpu
from jax.experimental.pallas import tpu_sc as plsc

_INFO = plsc.get_sparse_core_info()
_NC, _NS = _INFO.num_cores, _INFO.num_subcores
_NW = _NC * _NS


def _embed_gather(table_hbm, idx_hbm, out_hbm,
                  idx00, idx01, idx10, idx11,
                  r128_0, r128_1, r64_0, r64_1,
                  sem_i00, sem_i01, sem_i10, sem_i11,
                  sem_g0, sem_g1, sem_s0, sem_s1,
                  *, b_per_w, chunk):
    wid = lax.axis_index("s") * _NC + lax.axis_index("c")
    base_w = wid * b_per_w
    n_chunks = b_per_w // chunk
    n_pairs = n_chunks // 2
    idx_v = ((idx00, idx01), (idx10, idx11))
    sem_i = ((sem_i00, sem_i01), (sem_i10, sem_i11))
    r128 = (r128_0, r128_1)
    r64 = (r64_0, r64_1)
    sem_g = (sem_g0, sem_g1)
    sem_s = (sem_s0, sem_s1)

    def chunk_slice(g):
        return pl.ds(pl.multiple_of(base_w + g * chunk, 8), chunk)

    def idx_issue(p, slot, b):
        pltpu.async_copy(idx_hbm.at[chunk_slice(2 * p + b)],
                         idx_v[slot][b], sem_i[slot][b])

    def idx_wait(p, slot, b):
        pltpu.make_async_copy(idx_hbm.at[chunk_slice(2 * p + b)],
                              idx_v[slot][b], sem_i[slot][b]).wait()

    def compact(b):
        def row(i, carry):
            for j in range(4):
                sl = pl.ds(j * 16, 16)
                r64[b][i, sl] = r128[b][i, sl]
            return carry
        lax.fori_loop(0, chunk, row, 0, unroll=8)

    def do_pair(p, slot, first):
        for b in range(2):
            g = 2 * p + b

            @pl.when(jnp.logical_not(first))
            def _wait_prev_store():
                pltpu.make_async_copy(
                    r64[b], out_hbm.at[chunk_slice(g), :], sem_s[b]
                ).wait()

            idx_wait(p, slot, b)
            pltpu.async_copy(table_hbm.at[idx_v[slot][b]], r128[b], sem_g[b])
        for b in range(2):
            g = 2 * p + b
            pltpu.make_async_copy(
                table_hbm.at[idx_v[slot][b]], r128[b], sem_g[b]
            ).wait()
            compact(b)
            pltpu.async_copy(r64[b], out_hbm.at[chunk_slice(g), :], sem_s[b])
        @pl.when(p + 2 < n_pairs)
        def _prefetch():
            for b in range(2):
                idx_issue(p + 2, slot, b)

    for b in range(2):
        idx_issue(0, 0, b)
    for b in range(2):
        idx_issue(1, 1, b)

    def body(k, carry):
        do_pair(2 * k, 0, first=(k == 0))
        do_pair(2 * k + 1, 1, first=False)
        return carry

    lax.fori_loop(0, n_pairs // 2, body, 0)
    for b in range(2):
        g = n_chunks - 2 + b
        pltpu.make_async_copy(
            r64[b], out_hbm.at[chunk_slice(g), :], sem_s[b]
        ).wait()


def kernel(input_ids, embed_table):
    B, S = input_ids.shape
    V, D = embed_table.shape
    n = B * S
    assert n % _NW == 0
    b_per_w = n // _NW
    chunk = 200
    assert b_per_w % (4 * chunk) == 0

    idx_flat = input_ids.reshape(n)
    one = lax.optimization_barrier(jnp.float32(1.0))
    padded = jnp.pad(embed_table, ((0, 0), (0, 128 - D))) * one

    mesh = plsc.VectorSubcoreMesh(core_axis_name="c", subcore_axis_name="s")
    k = pl.kernel(
        functools.partial(_embed_gather, b_per_w=b_per_w, chunk=chunk),
        mesh=mesh,
        out_type=jax.ShapeDtypeStruct((n, D), jnp.float32),
        scratch_types=[
            pltpu.VMEM((chunk,), jnp.int32),
            pltpu.VMEM((chunk,), jnp.int32),
            pltpu.VMEM((chunk,), jnp.int32),
            pltpu.VMEM((chunk,), jnp.int32),
            pltpu.VMEM((chunk, 128), jnp.float32),
            pltpu.VMEM((chunk, 128), jnp.float32),
            pltpu.VMEM((chunk, D), jnp.float32),
            pltpu.VMEM((chunk, D), jnp.float32),
            pltpu.SemaphoreType.DMA,
            pltpu.SemaphoreType.DMA,
            pltpu.SemaphoreType.DMA,
            pltpu.SemaphoreType.DMA,
            pltpu.SemaphoreType.DMA,
            pltpu.SemaphoreType.DMA,
            pltpu.SemaphoreType.DMA,
            pltpu.SemaphoreType.DMA,
        ],
        compiler_params=pltpu.CompilerParams(use_tc_tiling_on_sc=True),
    )
    out = k(padded, idx_flat)
    return out.reshape(B, S, D)
